# Initial kernel scaffold; baseline (speedup 1.0000x reference)
import jax
import jax.numpy as jnp
from jax.experimental import pallas as pl


def kernel(x, edge_index, W1, b1, W2, b2, Wc, bc):
    raise NotImplementedError("write your pallas kernel here")


# speedup vs baseline: 15.2355x; 15.2355x over previous
import functools

import jax
import jax.numpy as jnp
from jax import lax
from jax.experimental import pallas as pl
from jax.experimental.pallas import tpu as pltpu
from jax.experimental.pallas import tpu_sc as plsc

N = 10000
NPAD = 10240
F = 128
HID = 512
NC, NS, L = 2, 16, 16
NW = NC * NS
B = 128
SR = NPAD // NS
MT = 256
GRID_M = NPAD // MT


def _vmesh():
    return plsc.VectorSubcoreMesh(
        core_axis_name="c", subcore_axis_name="s",
        num_cores=NC, num_subcores=NS)


def _deg_body(dst_hbm, out_hbm, dstv, degl, tmp, acc, shared):
    nblk = dst_hbm.shape[1]
    c = lax.axis_index("c")
    s = lax.axis_index("s")
    w = s * NC + c

    pltpu.sync_copy(dst_hbm.at[w], dstv)

    def zero_deg(i, _):
        degl[pl.ds(i * L, L)] = jnp.zeros((L,), jnp.float32)
        return 0
    lax.fori_loop(0, NPAD // L, zero_deg, 0)

    ones16 = jnp.ones((L,), jnp.float32)

    def blk(j, _):
        for k in range(B // L):
            idx = dstv[j, pl.ds(k * L, L)]
            plsc.addupdate_scatter(degl, [idx], ones16)
        return 0
    lax.fori_loop(0, nblk, blk, 0)

    pltpu.sync_copy(degl, shared.at[s])
    plsc.subcore_barrier()

    def zero_acc(i, _):
        acc[pl.ds(i * L, L)] = jnp.zeros((L,), jnp.float32)
        return 0
    lax.fori_loop(0, SR // L, zero_acc, 0)

    def red(t, _):
        pltpu.sync_copy(shared.at[t, pl.ds(s * SR, SR)], tmp)

        def add(i, __):
            acc[pl.ds(i * L, L)] = acc[pl.ds(i * L, L)] + tmp[pl.ds(i * L, L)]
            return 0
        lax.fori_loop(0, SR // L, add, 0)
        return 0
    lax.fori_loop(0, NS, red, 0)

    pltpu.sync_copy(acc, out_hbm.at[c, pl.ds(s * SR, SR)])


def _deg_halves(dst_t):
    nblk = dst_t.shape[1]
    return pl.kernel(
        _deg_body,
        out_type=jax.ShapeDtypeStruct((NC, NPAD), jnp.float32),
        mesh=_vmesh(),
        compiler_params=pltpu.CompilerParams(needs_layout_passes=False),
        scratch_types=[
            pltpu.VMEM((nblk, B), jnp.int32),
            pltpu.VMEM((NPAD,), jnp.float32),
            pltpu.VMEM((SR,), jnp.float32),
            pltpu.VMEM((SR,), jnp.float32),
            pltpu.VMEM_SHARED((NS, NPAD), jnp.float32),
        ],
    )(dst_t)


def _agg_body(table, srcb, dstb, out, srcv, dstv, rows, sem, shared):
    nblk = srcb.shape[1]
    c = lax.axis_index("c")
    s = lax.axis_index("s")
    w = s * NC + c

    pltpu.sync_copy(srcb.at[w], srcv)
    pltpu.sync_copy(dstb.at[w], dstv)

    @pl.when(c == 0)
    def _():
        pltpu.sync_copy(table.at[pl.ds(s * SR, SR)],
                        shared.at[pl.ds(s * SR, SR)])

    @pl.when(c == 1)
    def _():
        def zr(i, _):
            for k in range(F // L):
                rows[i, pl.ds(k * L, L)] = jnp.zeros((L,), jnp.float32)
            return 0
        lax.fori_loop(0, B, zr, 0)
        for t in range(SR // B):
            pltpu.sync_copy(rows, shared.at[pl.ds(s * SR + t * B, B)])

    plsc.subcore_barrier()

    def blk(j, _):
        pltpu.async_copy(table.at[srcv.at[j]], rows, sem).wait()
        pltpu.sync_copy(rows, shared.at[dstv.at[j]], add=True)
        return 0
    lax.fori_loop(0, nblk, blk, 0)

    plsc.subcore_barrier()
    pltpu.sync_copy(shared.at[pl.ds(s * SR, SR)],
                    out.at[c, pl.ds(s * SR, SR)])


def _agg_halves(table, src_t, dst_t):
    nblk = src_t.shape[1]
    return pl.kernel(
        _agg_body,
        out_type=jax.ShapeDtypeStruct((NC, NPAD, F), jnp.float32),
        mesh=_vmesh(),
        compiler_params=pltpu.CompilerParams(needs_layout_passes=False),
        scratch_types=[
            pltpu.VMEM((nblk, B), jnp.int32),
            pltpu.VMEM((nblk, B), jnp.int32),
            pltpu.VMEM((B, F), jnp.float32),
            pltpu.SemaphoreType.DMA,
            pltpu.VMEM_SHARED((NPAD, F), jnp.float32),
        ],
    )(table, src_t, dst_t)


def _dinv_tile(i, d0_ref, d1_ref):
    rows = i * MT + lax.broadcasted_iota(jnp.int32, (MT, 1), 0)
    deg = d0_ref[...] + d1_ref[...] + 1.0
    return jnp.where(rows < N, lax.rsqrt(deg), 0.0)


def _scale_body(x_ref, d0_ref, d1_ref, o_ref):
    o_ref[...] = x_ref[...] * _dinv_tile(pl.program_id(0), d0_ref, d1_ref)


def _scale(x_pad, d0, d1):
    return pl.pallas_call(
        _scale_body,
        grid=(GRID_M,),
        in_specs=[
            pl.BlockSpec((MT, F), lambda i: (i, 0)),
            pl.BlockSpec((MT, 1), lambda i: (i, 0)),
            pl.BlockSpec((MT, 1), lambda i: (i, 0)),
        ],
        out_specs=pl.BlockSpec((MT, F), lambda i: (i, 0)),
        out_shape=jax.ShapeDtypeStruct((NPAD, F), jnp.float32),
    )(x_pad, d0, d1)


def _mm1_body(a0, a1, d0, d1, w_ref, b_ref, o_ref):
    av = (a0[...] + a1[...]) * _dinv_tile(pl.program_id(0), d0, d1)
    h = jnp.dot(av, w_ref[...], preferred_element_type=jnp.float32)
    o_ref[...] = jnp.maximum(h + b_ref[...], 0.0)


def _mm1(a0, a1, d0, d1, W1, b1r):
    return pl.pallas_call(
        _mm1_body,
        grid=(GRID_M,),
        in_specs=[
            pl.BlockSpec((MT, F), lambda i: (i, 0)),
            pl.BlockSpec((MT, F), lambda i: (i, 0)),
            pl.BlockSpec((MT, 1), lambda i: (i, 0)),
            pl.BlockSpec((MT, 1), lambda i: (i, 0)),
            pl.BlockSpec((F, HID), lambda i: (0, 0)),
            pl.BlockSpec((1, HID), lambda i: (0, 0)),
        ],
        out_specs=pl.BlockSpec((MT, HID), lambda i: (i, 0)),
        out_shape=jax.ShapeDtypeStruct((NPAD, HID), jnp.float32),
    )(a0, a1, d0, d1, W1, b1r)


def _mm2_body(h_ref, d0, d1, w_ref, o_ref):
    y = jnp.dot(h_ref[...], w_ref[...], preferred_element_type=jnp.float32)
    o_ref[...] = y * _dinv_tile(pl.program_id(0), d0, d1)


def _mm2(h1, d0, d1, W2):
    return pl.pallas_call(
        _mm2_body,
        grid=(GRID_M,),
        in_specs=[
            pl.BlockSpec((MT, HID), lambda i: (i, 0)),
            pl.BlockSpec((MT, 1), lambda i: (i, 0)),
            pl.BlockSpec((MT, 1), lambda i: (i, 0)),
            pl.BlockSpec((HID, F), lambda i: (0, 0)),
        ],
        out_specs=pl.BlockSpec((MT, F), lambda i: (i, 0)),
        out_shape=jax.ShapeDtypeStruct((NPAD, F), jnp.float32),
    )(h1, d0, d1, W2)


def _final_body(a0, a1, d0, d1, b2_ref, wc_ref, bc_ref, o_ref, acc_ref):
    i = pl.program_id(0)
    rows = i * MT + lax.broadcasted_iota(jnp.int32, (MT, 1), 0)
    h2 = (a0[...] + a1[...]) * _dinv_tile(i, d0, d1) + b2_ref[...]
    h2 = jnp.where(rows < N, jnp.maximum(h2, 0.0), 0.0)

    @pl.when(i == 0)
    def _():
        acc_ref[...] = jnp.zeros_like(acc_ref)

    acc_ref[...] += jnp.sum(h2, axis=0, keepdims=True)

    @pl.when(i == GRID_M - 1)
    def _():
        g = acc_ref[...] * (1.0 / N)
        logits = jnp.dot(g, wc_ref[...], preferred_element_type=jnp.float32)
        o_ref[...] = jax.nn.sigmoid(logits + bc_ref[...])


def _final(a0, a1, d0, d1, b2r, wc_pad, bc_pad):
    return pl.pallas_call(
        _final_body,
        grid=(GRID_M,),
        in_specs=[
            pl.BlockSpec((MT, F), lambda i: (i, 0)),
            pl.BlockSpec((MT, F), lambda i: (i, 0)),
            pl.BlockSpec((MT, 1), lambda i: (i, 0)),
            pl.BlockSpec((MT, 1), lambda i: (i, 0)),
            pl.BlockSpec((1, F), lambda i: (0, 0)),
            pl.BlockSpec((F, F), lambda i: (0, 0)),
            pl.BlockSpec((1, F), lambda i: (0, 0)),
        ],
        out_specs=pl.BlockSpec((1, F), lambda i: (0, 0)),
        out_shape=jax.ShapeDtypeStruct((1, F), jnp.float32),
        scratch_shapes=[pltpu.VMEM((1, F), jnp.float32)],
    )(a0, a1, d0, d1, b2r, wc_pad, bc_pad)


def kernel(x, edge_index, W1, b1, W2, b2, Wc, bc):
    E = edge_index.shape[1]
    nblk = -(-E // (NW * B))
    epad = NW * nblk * B

    src = edge_index[0].astype(jnp.int32)
    dst = edge_index[1].astype(jnp.int32)
    pad = jnp.full((epad - E,), N, jnp.int32)
    src_t = jnp.concatenate([src, pad]).reshape(NW, nblk, B)
    dst_t = jnp.concatenate([dst, pad]).reshape(NW, nblk, B)

    x_pad = jnp.zeros((NPAD, F), jnp.float32).at[:N].set(x)

    deg = _deg_halves(dst_t)
    d0 = deg[0].reshape(NPAD, 1)
    d1 = deg[1].reshape(NPAD, 1)

    xs = _scale(x_pad, d0, d1)
    a = _agg_halves(xs, src_t, dst_t)
    h1 = _mm1(a[0], a[1], d0, d1, W1, b1.reshape(1, HID))
    ys2 = _mm2(h1, d0, d1, W2)
    b = _agg_halves(ys2, src_t, dst_t)

    wc_pad = jnp.zeros((F, F), jnp.float32).at[:, :10].set(Wc)
    bc_pad = jnp.zeros((1, F), jnp.float32).at[0, :10].set(bc)
    out = _final(b[0], b[1], d0, d1, b2.reshape(1, F), wc_pad, bc_pad)
    return out[:, :10]


# speedup vs baseline: 21.4847x; 1.4102x over previous
import functools

import jax
import jax.numpy as jnp
from jax import lax
from jax.experimental import pallas as pl
from jax.experimental.pallas import tpu as pltpu
from jax.experimental.pallas import tpu_sc as plsc

N = 10000
NPAD = 10240
F = 128
HID = 512
NC, NS, L = 2, 16, 16
NW = NC * NS
B = 112
SR = NPAD // NS
MT = 256
GRID_M = NPAD // MT


def _vmesh():
    return plsc.VectorSubcoreMesh(
        core_axis_name="c", subcore_axis_name="s",
        num_cores=NC, num_subcores=NS)


def _deg_body(dst_hbm, out_hbm, dstv, degl, tmp, acc, shared):
    nblk = dst_hbm.shape[1]
    c = lax.axis_index("c")
    s = lax.axis_index("s")
    w = s * NC + c

    pltpu.sync_copy(dst_hbm.at[w], dstv)

    def zero_deg(i, _):
        degl[pl.ds(i * L, L)] = jnp.zeros((L,), jnp.float32)
        return 0
    lax.fori_loop(0, NPAD // L, zero_deg, 0)

    ones16 = jnp.ones((L,), jnp.float32)

    def blk(j, _):
        for k in range(B // L):
            idx = dstv[j, pl.ds(k * L, L)]
            plsc.addupdate_scatter(degl, [idx], ones16)
        return 0
    lax.fori_loop(0, nblk, blk, 0)

    pltpu.sync_copy(degl, shared.at[s])
    plsc.subcore_barrier()

    def zero_acc(i, _):
        acc[pl.ds(i * L, L)] = jnp.zeros((L,), jnp.float32)
        return 0
    lax.fori_loop(0, SR // L, zero_acc, 0)

    def red(t, _):
        pltpu.sync_copy(shared.at[t, pl.ds(s * SR, SR)], tmp)

        def add(i, __):
            acc[pl.ds(i * L, L)] = acc[pl.ds(i * L, L)] + tmp[pl.ds(i * L, L)]
            return 0
        lax.fori_loop(0, SR // L, add, 0)
        return 0
    lax.fori_loop(0, NS, red, 0)

    pltpu.sync_copy(acc, out_hbm.at[c, pl.ds(s * SR, SR)])


def _deg_halves(dst_t):
    nblk = dst_t.shape[1]
    return pl.kernel(
        _deg_body,
        out_type=jax.ShapeDtypeStruct((NC, NPAD), jnp.float32),
        mesh=_vmesh(),
        compiler_params=pltpu.CompilerParams(needs_layout_passes=False),
        scratch_types=[
            pltpu.VMEM((nblk, B), jnp.int32),
            pltpu.VMEM((NPAD,), jnp.float32),
            pltpu.VMEM((SR,), jnp.float32),
            pltpu.VMEM((SR,), jnp.float32),
            pltpu.VMEM_SHARED((NS, NPAD), jnp.float32),
        ],
    )(dst_t)


NBUF = 2


def _agg_body(table, pkb, out, pkv, sbuf, dbuf, rows, sem0, sem1, shared):
    nblk = pkb.shape[1]
    sems = (sem0, sem1)
    c = lax.axis_index("c")
    s = lax.axis_index("s")
    w = s * NC + c

    pltpu.sync_copy(pkb.at[w], pkv)

    def unpack(j, b):
        for k in range(B // L):
            v = pkv[j, pl.ds(k * L, L)]
            dbuf[b, pl.ds(k * L, L)] = lax.shift_right_logical(v, 16)
            sbuf[b, pl.ds(k * L, L)] = lax.bitwise_and(v, 0xFFFF)

    @pl.when(c == 0)
    def _():
        pltpu.sync_copy(table.at[pl.ds(s * SR, SR)],
                        shared.at[pl.ds(s * SR, SR)])

    @pl.when(c == 1)
    def _():
        zch = 80
        def zr(i, _):
            for k in range(F // L):
                rows[0, i, pl.ds(k * L, L)] = jnp.zeros((L,), jnp.float32)
            return 0
        lax.fori_loop(0, zch, zr, 0)
        for t in range(SR // zch):
            pltpu.sync_copy(rows.at[0, pl.ds(0, zch)],
                            shared.at[pl.ds(s * SR + t * zch, zch)])

    plsc.subcore_barrier()

    for b in range(NBUF):
        unpack(b, b)
        pltpu.async_copy(table.at[sbuf.at[b]], rows.at[b], sems[b])

    def blk(g, _):
        for b in range(NBUF):
            j = g * NBUF + b
            pltpu.make_async_copy(table.at[pl.ds(0, B)], rows.at[b],
                                  sems[b]).wait()
            pltpu.sync_copy(rows.at[b], shared.at[dbuf.at[b]], add=True)

            @pl.when(j + NBUF < nblk)
            def _():
                unpack(j + NBUF, b)
                pltpu.async_copy(table.at[sbuf.at[b]], rows.at[b],
                                 sems[b])
        return 0
    lax.fori_loop(0, nblk // NBUF, blk, 0)

    plsc.subcore_barrier()
    pltpu.sync_copy(shared.at[pl.ds(s * SR, SR)],
                    out.at[c, pl.ds(s * SR, SR)])


def _agg_halves(table, pk_t):
    nblk = pk_t.shape[1]
    return pl.kernel(
        _agg_body,
        out_type=jax.ShapeDtypeStruct((NC, NPAD, F), jnp.float32),
        mesh=_vmesh(),
        compiler_params=pltpu.CompilerParams(needs_layout_passes=False),
        scratch_types=[
            pltpu.VMEM((nblk, B), jnp.int32),
            pltpu.VMEM((NBUF, B), jnp.int32),
            pltpu.VMEM((NBUF, B), jnp.int32),
            pltpu.VMEM((NBUF, B, F), jnp.float32),
            pltpu.SemaphoreType.DMA,
            pltpu.SemaphoreType.DMA,
            pltpu.VMEM_SHARED((NPAD, F), jnp.float32),
        ],
    )(table, pk_t)


def _dinv_tile(i, d0_ref, d1_ref):
    rows = i * MT + lax.broadcasted_iota(jnp.int32, (MT, 1), 0)
    deg = d0_ref[...] + d1_ref[...] + 1.0
    return jnp.where(rows < N, lax.rsqrt(deg), 0.0)


def _scale_body(x_ref, d0_ref, d1_ref, o_ref):
    o_ref[...] = x_ref[...] * _dinv_tile(pl.program_id(0), d0_ref, d1_ref)


def _scale(x_pad, d0, d1):
    return pl.pallas_call(
        _scale_body,
        grid=(GRID_M,),
        in_specs=[
            pl.BlockSpec((MT, F), lambda i: (i, 0)),
            pl.BlockSpec((MT, 1), lambda i: (i, 0)),
            pl.BlockSpec((MT, 1), lambda i: (i, 0)),
        ],
        out_specs=pl.BlockSpec((MT, F), lambda i: (i, 0)),
        out_shape=jax.ShapeDtypeStruct((NPAD, F), jnp.float32),
    )(x_pad, d0, d1)


def _mm1_body(a0, a1, d0, d1, w_ref, b_ref, o_ref):
    av = (a0[...] + a1[...]) * _dinv_tile(pl.program_id(0), d0, d1)
    h = jnp.dot(av, w_ref[...], preferred_element_type=jnp.float32)
    o_ref[...] = jnp.maximum(h + b_ref[...], 0.0)


def _mm1(a0, a1, d0, d1, W1, b1r):
    return pl.pallas_call(
        _mm1_body,
        grid=(GRID_M,),
        in_specs=[
            pl.BlockSpec((MT, F), lambda i: (i, 0)),
            pl.BlockSpec((MT, F), lambda i: (i, 0)),
            pl.BlockSpec((MT, 1), lambda i: (i, 0)),
            pl.BlockSpec((MT, 1), lambda i: (i, 0)),
            pl.BlockSpec((F, HID), lambda i: (0, 0)),
            pl.BlockSpec((1, HID), lambda i: (0, 0)),
        ],
        out_specs=pl.BlockSpec((MT, HID), lambda i: (i, 0)),
        out_shape=jax.ShapeDtypeStruct((NPAD, HID), jnp.float32),
    )(a0, a1, d0, d1, W1, b1r)


def _mm2_body(h_ref, d0, d1, w_ref, o_ref):
    y = jnp.dot(h_ref[...], w_ref[...], preferred_element_type=jnp.float32)
    o_ref[...] = y * _dinv_tile(pl.program_id(0), d0, d1)


def _mm2(h1, d0, d1, W2):
    return pl.pallas_call(
        _mm2_body,
        grid=(GRID_M,),
        in_specs=[
            pl.BlockSpec((MT, HID), lambda i: (i, 0)),
            pl.BlockSpec((MT, 1), lambda i: (i, 0)),
            pl.BlockSpec((MT, 1), lambda i: (i, 0)),
            pl.BlockSpec((HID, F), lambda i: (0, 0)),
        ],
        out_specs=pl.BlockSpec((MT, F), lambda i: (i, 0)),
        out_shape=jax.ShapeDtypeStruct((NPAD, F), jnp.float32),
    )(h1, d0, d1, W2)


def _final_body(a0, a1, d0, d1, b2_ref, wc_ref, bc_ref, o_ref, acc_ref):
    i = pl.program_id(0)
    rows = i * MT + lax.broadcasted_iota(jnp.int32, (MT, 1), 0)
    h2 = (a0[...] + a1[...]) * _dinv_tile(i, d0, d1) + b2_ref[...]
    h2 = jnp.where(rows < N, jnp.maximum(h2, 0.0), 0.0)

    @pl.when(i == 0)
    def _():
        acc_ref[...] = jnp.zeros_like(acc_ref)

    acc_ref[...] += jnp.sum(h2, axis=0, keepdims=True)

    @pl.when(i == GRID_M - 1)
    def _():
        g = acc_ref[...] * (1.0 / N)
        logits = jnp.dot(g, wc_ref[...], preferred_element_type=jnp.float32)
        o_ref[...] = jax.nn.sigmoid(logits + bc_ref[...])


def _final(a0, a1, d0, d1, b2r, wc_pad, bc_pad):
    return pl.pallas_call(
        _final_body,
        grid=(GRID_M,),
        in_specs=[
            pl.BlockSpec((MT, F), lambda i: (i, 0)),
            pl.BlockSpec((MT, F), lambda i: (i, 0)),
            pl.BlockSpec((MT, 1), lambda i: (i, 0)),
            pl.BlockSpec((MT, 1), lambda i: (i, 0)),
            pl.BlockSpec((1, F), lambda i: (0, 0)),
            pl.BlockSpec((F, F), lambda i: (0, 0)),
            pl.BlockSpec((1, F), lambda i: (0, 0)),
        ],
        out_specs=pl.BlockSpec((1, F), lambda i: (0, 0)),
        out_shape=jax.ShapeDtypeStruct((1, F), jnp.float32),
        scratch_shapes=[pltpu.VMEM((1, F), jnp.float32)],
    )(a0, a1, d0, d1, b2r, wc_pad, bc_pad)


def kernel(x, edge_index, W1, b1, W2, b2, Wc, bc):
    E = edge_index.shape[1]
    nblk = -(-E // (NW * B))
    nblk = -(-nblk // NBUF) * NBUF
    epad = NW * nblk * B

    src = edge_index[0].astype(jnp.int32)
    dst = edge_index[1].astype(jnp.int32)
    pad = jnp.full((epad - E,), N, jnp.int32)
    dst_t = jnp.concatenate([dst, pad]).reshape(NW, nblk, B)
    pk = jnp.concatenate([src, pad]) | (jnp.concatenate([dst, pad]) << 16)
    pk_t = pk.reshape(NW, nblk, B)

    x_pad = jnp.zeros((NPAD, F), jnp.float32).at[:N].set(x)

    deg = _deg_halves(dst_t)
    d0 = deg[0].reshape(NPAD, 1)
    d1 = deg[1].reshape(NPAD, 1)

    xs = _scale(x_pad, d0, d1)
    a = _agg_halves(xs, pk_t)
    h1 = _mm1(a[0], a[1], d0, d1, W1, b1.reshape(1, HID))
    ys2 = _mm2(h1, d0, d1, W2)
    b = _agg_halves(ys2, pk_t)

    wc_pad = jnp.zeros((F, F), jnp.float32).at[:, :10].set(Wc)
    bc_pad = jnp.zeros((1, F), jnp.float32).at[0, :10].set(bc)
    out = _final(b[0], b[1], d0, d1, b2.reshape(1, F), wc_pad, bc_pad)
    return out[:, :10]
